# Initial kernel scaffold; baseline (speedup 1.0000x reference)
import jax
import jax.numpy as jnp
from jax.experimental import pallas as pl


def kernel(x, edge_index, edge_attr, We1, be1, We2, be2, Wn1, bn1, Wn2, bn2):
    raise NotImplementedError("write your pallas kernel here")


# speedup vs baseline: 2.8808x; 2.8808x over previous
import functools

import jax
import jax.numpy as jnp
from jax import lax
from jax.experimental import pallas as pl
from jax.experimental.pallas import tpu as pltpu
from jax.experimental.pallas import tpu_sc as plsc

N = 10000
E = 320000
H = 128
L = 16
NC = 2
NS = 16
NW = NC * NS
EPW = E // NW
CH = 80
NCH = EPW // CH
NP = 10240
RPT = NP // NS
HL = H // L

@functools.cache
def _mesh():
    return plsc.VectorSubcoreMesh(
        core_axis_name="c", subcore_axis_name="s", num_cores=NC, num_subcores=NS
    )


def _gather_body(p_hbm, q_hbm, si_hbm, di_hbm, t_hbm, sidx, didx, bufp, bufq,
                 sem):
    c = lax.axis_index("c")
    s = lax.axis_index("s")
    wid = s * NC + c
    base = wid * EPW
    pltpu.sync_copy(si_hbm.at[wid], sidx)
    pltpu.sync_copy(di_hbm.at[wid], didx)

    def chunk(ch, carry):
        pltpu.async_copy(p_hbm.at[sidx.at[ch]], bufp, sem).wait()
        pltpu.async_copy(q_hbm.at[didx.at[ch]], bufq, sem).wait()

        def row(i, carry2):
            for j in range(HL):
                sl = pl.ds(j * L, L)
                bufp[i, sl] = bufp[i, sl] + bufq[i, sl]
            return carry2

        lax.fori_loop(0, CH, row, 0)
        off = pl.multiple_of(base + ch * CH, 8)
        pltpu.sync_copy(bufp, t_hbm.at[pl.ds(off, CH)])
        return carry

    lax.fori_loop(0, NCH, chunk, 0)


@functools.cache
def _gather_t():
    return pl.kernel(
        _gather_body,
        out_type=jax.ShapeDtypeStruct((E, H), jnp.float32),
        mesh=_mesh(),
        scratch_types=[
            pltpu.VMEM((NCH, CH), jnp.int32),
            pltpu.VMEM((NCH, CH), jnp.int32),
            pltpu.VMEM((CH, H), jnp.float32),
            pltpu.VMEM((CH, H), jnp.float32),
            pltpu.SemaphoreType.DMA,
        ],
    )


def _scatter_body(e_hbm, di_hbm, out_hbm, didx, rows, zbuf, shared, sem):
    c = lax.axis_index("c")
    s = lax.axis_index("s")
    wid = s * NC + c
    base = wid * EPW

    def zrow(i, carry):
        for j in range(HL):
            zbuf[i, pl.ds(j * L, L)] = jnp.zeros((L,), jnp.float32)
        return carry

    lax.fori_loop(0, CH, zrow, 0)
    my_off = pl.multiple_of(s * RPT, 8)
    for r in range(RPT // CH):
        pltpu.sync_copy(zbuf, shared.at[pl.ds(my_off + r * CH, CH)])
    plsc.subcore_barrier()

    pltpu.sync_copy(di_hbm.at[wid], didx)

    def chunk(ch, carry):
        off = pl.multiple_of(base + ch * CH, 8)
        pltpu.sync_copy(e_hbm.at[pl.ds(off, CH)], rows)
        pltpu.sync_copy(rows, shared.at[didx.at[ch]], add=True)
        return carry

    lax.fori_loop(0, NCH, chunk, 0)
    plsc.subcore_barrier()

    for r in range(RPT // CH):
        pltpu.sync_copy(shared.at[pl.ds(my_off + r * CH, CH)], zbuf)
        pltpu.sync_copy(zbuf, out_hbm.at[c, pl.ds(my_off + r * CH, CH)])


@functools.cache
def _scatter_agg():
    return pl.kernel(
        _scatter_body,
        out_type=jax.ShapeDtypeStruct((NC, NP, H), jnp.float32),
        mesh=_mesh(),
        scratch_types=[
            pltpu.VMEM((NCH, CH), jnp.int32),
            pltpu.VMEM((CH, H), jnp.float32),
            pltpu.VMEM((CH, H), jnp.float32),
            pltpu.VMEM_SHARED((NP, H), jnp.float32),
            pltpu.SemaphoreType.DMA,
        ],
    )


def _xform_body(x_ref, a_ref, b_ref, be1_ref, p_ref, q_ref):
    xb = x_ref[...]
    p_ref[...] = (
        jnp.dot(xb, a_ref[...], preferred_element_type=jnp.float32)
        + be1_ref[...]
    )
    q_ref[...] = jnp.dot(xb, b_ref[...], preferred_element_type=jnp.float32)


def _tc_xform(x, a, b, be1s):
    g = 5
    rb = N // g
    return pl.pallas_call(
        _xform_body,
        grid=(g,),
        in_specs=[
            pl.BlockSpec((rb, H), lambda i: (i, 0)),
            pl.BlockSpec((H, H), lambda i: (0, 0)),
            pl.BlockSpec((H, H), lambda i: (0, 0)),
            pl.BlockSpec((1, H), lambda i: (0, 0)),
        ],
        out_specs=[pl.BlockSpec((rb, H), lambda i: (i, 0))] * 2,
        out_shape=[jax.ShapeDtypeStruct((N, H), jnp.float32)] * 2,
    )(x, a, b, be1s.reshape(1, H))


def _edge_body(t_ref, e_ref, c_ref, w2_ref, be2_ref, o_ref):
    eb = e_ref[...]
    h = jnp.maximum(
        t_ref[...] + jnp.dot(eb, c_ref[...], preferred_element_type=jnp.float32),
        0.0,
    )
    o_ref[...] = (
        eb
        + jnp.dot(h, w2_ref[...], preferred_element_type=jnp.float32)
        + be2_ref[...]
    )


def _edge_mlp(t, e, c, w2, be2s):
    g = 125
    rb = E // g
    return pl.pallas_call(
        _edge_body,
        grid=(g,),
        in_specs=[
            pl.BlockSpec((rb, H), lambda i: (i, 0)),
            pl.BlockSpec((rb, H), lambda i: (i, 0)),
            pl.BlockSpec((H, H), lambda i: (0, 0)),
            pl.BlockSpec((H, H), lambda i: (0, 0)),
            pl.BlockSpec((1, H), lambda i: (0, 0)),
        ],
        out_specs=pl.BlockSpec((rb, H), lambda i: (i, 0)),
        out_shape=jax.ShapeDtypeStruct((E, H), jnp.float32),
    )(t, e, c, w2, be2s.reshape(1, H))


def _node_body(x_ref, pa_ref, w1_ref, bn1_ref, w2_ref, bn2_ref, o_ref):
    xb = x_ref[...]
    agg = pa_ref[0] + pa_ref[1]
    w1 = w1_ref[...]
    h = jnp.maximum(
        jnp.dot(xb, w1[:H], preferred_element_type=jnp.float32)
        + jnp.dot(agg, w1[H:], preferred_element_type=jnp.float32)
        + bn1_ref[...],
        0.0,
    )
    o_ref[...] = (
        xb
        + jnp.dot(h, w2_ref[...], preferred_element_type=jnp.float32)
        + bn2_ref[...]
    )


def _node_mlp(x, parts, w1, bn1s, w2, bn2s):
    g = 5
    rb = N // g
    return pl.pallas_call(
        _node_body,
        grid=(g,),
        in_specs=[
            pl.BlockSpec((rb, H), lambda i: (i, 0)),
            pl.BlockSpec((NC, rb, H), lambda i: (0, i, 0)),
            pl.BlockSpec((2 * H, H), lambda i: (0, 0)),
            pl.BlockSpec((1, H), lambda i: (0, 0)),
            pl.BlockSpec((H, H), lambda i: (0, 0)),
            pl.BlockSpec((1, H), lambda i: (0, 0)),
        ],
        out_specs=pl.BlockSpec((rb, H), lambda i: (i, 0)),
        out_shape=jax.ShapeDtypeStruct((N, H), jnp.float32),
    )(x, parts, w1, bn1s.reshape(1, H), w2, bn2s.reshape(1, H))


def kernel(x, edge_index, edge_attr, We1, be1, We2, be2, Wn1, bn1, Wn2, bn2):
    src3 = edge_index[0].reshape(NW, NCH, CH)
    dst3 = edge_index[1].reshape(NW, NCH, CH)
    e = edge_attr
    n_steps = We1.shape[0]
    for s in range(n_steps):
        a = We1[s, :H]
        b = We1[s, H : 2 * H]
        c = We1[s, 2 * H :]
        p, q = _tc_xform(x, a, b, be1[s])
        t = _gather_t()(p, q, src3, dst3)
        e = _edge_mlp(t, e, c, We2[s], be2[s])
        parts = _scatter_agg()(e, dst3)
        x = _node_mlp(x, parts, Wn1[s], bn1[s], Wn2[s], bn2[s])
    return x, e


# speedup vs baseline: 3.1027x; 1.0770x over previous
import functools

import jax
import jax.numpy as jnp
from jax import lax
from jax.experimental import pallas as pl
from jax.experimental.pallas import tpu as pltpu
from jax.experimental.pallas import tpu_sc as plsc

N = 10000
E = 320000
H = 128
L = 16
NC = 2
NS = 16
NW = NC * NS
EPW = E // NW
CH = 80
NCH = EPW // CH
NP = 10240
RPT = NP // NS
HL = H // L

@functools.cache
def _mesh():
    return plsc.VectorSubcoreMesh(
        core_axis_name="c", subcore_axis_name="s", num_cores=NC, num_subcores=NS
    )


GB = 2


def _gather_body(p_hbm, q_hbm, si_hbm, di_hbm, t_hbm, sidx, didx, bufp, bufq,
                 bufs, gsp, gsq, sss):
    c = lax.axis_index("c")
    s = lax.axis_index("s")
    wid = s * NC + c
    base = wid * EPW
    pltpu.sync_copy(si_hbm.at[wid], sidx)
    pltpu.sync_copy(di_hbm.at[wid], didx)

    def issue(ch, b):
        pltpu.async_copy(p_hbm.at[sidx.at[ch]], bufp.at[b], gsp.at[b])
        pltpu.async_copy(q_hbm.at[didx.at[ch]], bufq.at[b], gsq.at[b])

    issue(0, 0)
    issue(1, 1)

    def chunk(ch, carry):
        b = lax.rem(ch, GB)
        pltpu.make_async_copy(p_hbm.at[sidx.at[ch]], bufp.at[b],
                              gsp.at[b]).wait()
        pltpu.make_async_copy(q_hbm.at[didx.at[ch]], bufq.at[b],
                              gsq.at[b]).wait()

        @pl.when(ch >= GB)
        def _():
            off2 = pl.multiple_of(base + (ch - GB) * CH, 8)
            pltpu.make_async_copy(bufs.at[b], t_hbm.at[pl.ds(off2, CH)],
                                  sss.at[b]).wait()

        def row(i, carry2):
            for j in range(HL):
                sl = pl.ds(j * L, L)
                bufs[b, i, sl] = bufp[b, i, sl] + bufq[b, i, sl]
            return carry2

        lax.fori_loop(0, CH, row, 0, unroll=2)
        off = pl.multiple_of(base + ch * CH, 8)
        pltpu.async_copy(bufs.at[b], t_hbm.at[pl.ds(off, CH)], sss.at[b])

        @pl.when(ch + GB < NCH)
        def _():
            issue(ch + GB, b)

        return carry

    lax.fori_loop(0, NCH, chunk, 0)
    for k in range(NCH - GB, NCH):
        b_ = k % GB
        off = pl.multiple_of(base + k * CH, 8)
        pltpu.make_async_copy(bufs.at[b_], t_hbm.at[pl.ds(off, CH)],
                              sss.at[b_]).wait()


@functools.cache
def _gather_t():
    return pl.kernel(
        _gather_body,
        out_type=jax.ShapeDtypeStruct((E, H), jnp.float32),
        mesh=_mesh(),
        scratch_types=[
            pltpu.VMEM((NCH, CH), jnp.int32),
            pltpu.VMEM((NCH, CH), jnp.int32),
            pltpu.VMEM((GB, CH, H), jnp.float32),
            pltpu.VMEM((GB, CH, H), jnp.float32),
            pltpu.VMEM((GB, CH, H), jnp.float32),
            pltpu.SemaphoreType.DMA((GB,)),
            pltpu.SemaphoreType.DMA((GB,)),
            pltpu.SemaphoreType.DMA((GB,)),
        ],
    )


SB = 3
ZR = 16


def _scatter_body(e_hbm, di_hbm, out_hbm, ibuf, bufe, zbuf, shared, lsem,
                  csem, isem):
    c = lax.axis_index("c")
    s = lax.axis_index("s")
    wid = s * NC + c
    base = wid * EPW

    def zrow(i, carry):
        for j in range(HL):
            zbuf[i, pl.ds(j * L, L)] = jnp.zeros((L,), jnp.float32)
        return carry

    lax.fori_loop(0, ZR, zrow, 0)
    my_off = pl.multiple_of(s * RPT, 8)
    for r in range(RPT // ZR):
        pltpu.sync_copy(zbuf, shared.at[pl.ds(my_off + r * ZR, ZR)])
    plsc.subcore_barrier()

    def lissue(ch, b):
        off = pl.multiple_of(base + ch * CH, 8)
        pltpu.async_copy(di_hbm.at[wid, ch], ibuf.at[b], isem.at[b])
        pltpu.async_copy(e_hbm.at[pl.ds(off, CH)], bufe.at[b], lsem.at[b])

    lissue(0, 0)

    def chunk(ch, carry):
        b = lax.rem(ch, SB)
        off = pl.multiple_of(base + ch * CH, 8)
        pltpu.make_async_copy(di_hbm.at[wid, ch], ibuf.at[b],
                              isem.at[b]).wait()
        pltpu.make_async_copy(e_hbm.at[pl.ds(off, CH)], bufe.at[b],
                              lsem.at[b]).wait()

        nxt = ch + 1

        @pl.when(nxt < NCH)
        def _():
            b1 = lax.rem(nxt, SB)

            @pl.when(ch >= 2)
            def _():
                pltpu.make_async_copy(bufe.at[b1],
                                      shared.at[ibuf.at[b1]],
                                      csem.at[b1]).wait()

            lissue(nxt, b1)

        pltpu.async_copy(bufe.at[b], shared.at[ibuf.at[b]], csem.at[b],
                         add=True)
        return carry

    lax.fori_loop(0, NCH, chunk, 0)
    for k in range(NCH - SB, NCH):
        b_ = k % SB
        pltpu.make_async_copy(bufe.at[b_], shared.at[ibuf.at[b_]],
                              csem.at[b_]).wait()
    plsc.subcore_barrier()

    for r in range(RPT // ZR):
        pltpu.sync_copy(shared.at[pl.ds(my_off + r * ZR, ZR)], zbuf)
        pltpu.sync_copy(zbuf, out_hbm.at[c, pl.ds(my_off + r * ZR, ZR)])


@functools.cache
def _scatter_agg():
    return pl.kernel(
        _scatter_body,
        out_type=jax.ShapeDtypeStruct((NC, NP, H), jnp.float32),
        mesh=_mesh(),
        scratch_types=[
            pltpu.VMEM((SB, CH), jnp.int32),
            pltpu.VMEM((SB, CH, H), jnp.float32),
            pltpu.VMEM((ZR, H), jnp.float32),
            pltpu.VMEM_SHARED((NP, H), jnp.float32),
            pltpu.SemaphoreType.DMA((SB,)),
            pltpu.SemaphoreType.DMA((SB,)),
            pltpu.SemaphoreType.DMA((SB,)),
        ],
    )


def _xform_body(x_ref, a_ref, b_ref, be1_ref, p_ref, q_ref):
    xb = x_ref[...]
    p_ref[...] = (
        jnp.dot(xb, a_ref[...], preferred_element_type=jnp.float32)
        + be1_ref[...]
    )
    q_ref[...] = jnp.dot(xb, b_ref[...], preferred_element_type=jnp.float32)


def _tc_xform(x, a, b, be1s):
    g = 5
    rb = N // g
    return pl.pallas_call(
        _xform_body,
        grid=(g,),
        in_specs=[
            pl.BlockSpec((rb, H), lambda i: (i, 0)),
            pl.BlockSpec((H, H), lambda i: (0, 0)),
            pl.BlockSpec((H, H), lambda i: (0, 0)),
            pl.BlockSpec((1, H), lambda i: (0, 0)),
        ],
        out_specs=[pl.BlockSpec((rb, H), lambda i: (i, 0))] * 2,
        out_shape=[jax.ShapeDtypeStruct((N, H), jnp.float32)] * 2,
    )(x, a, b, be1s.reshape(1, H))


def _edge_body(t_ref, e_ref, c_ref, w2_ref, be2_ref, o_ref):
    eb = e_ref[...]
    h = jnp.maximum(
        t_ref[...] + jnp.dot(eb, c_ref[...], preferred_element_type=jnp.float32),
        0.0,
    )
    o_ref[...] = (
        eb
        + jnp.dot(h, w2_ref[...], preferred_element_type=jnp.float32)
        + be2_ref[...]
    )


def _edge_mlp(t, e, c, w2, be2s):
    g = 125
    rb = E // g
    return pl.pallas_call(
        _edge_body,
        grid=(g,),
        in_specs=[
            pl.BlockSpec((rb, H), lambda i: (i, 0)),
            pl.BlockSpec((rb, H), lambda i: (i, 0)),
            pl.BlockSpec((H, H), lambda i: (0, 0)),
            pl.BlockSpec((H, H), lambda i: (0, 0)),
            pl.BlockSpec((1, H), lambda i: (0, 0)),
        ],
        out_specs=pl.BlockSpec((rb, H), lambda i: (i, 0)),
        out_shape=jax.ShapeDtypeStruct((E, H), jnp.float32),
    )(t, e, c, w2, be2s.reshape(1, H))


def _node_body(x_ref, pa_ref, w1_ref, bn1_ref, w2_ref, bn2_ref, o_ref):
    xb = x_ref[...]
    agg = pa_ref[0] + pa_ref[1]
    w1 = w1_ref[...]
    h = jnp.maximum(
        jnp.dot(xb, w1[:H], preferred_element_type=jnp.float32)
        + jnp.dot(agg, w1[H:], preferred_element_type=jnp.float32)
        + bn1_ref[...],
        0.0,
    )
    o_ref[...] = (
        xb
        + jnp.dot(h, w2_ref[...], preferred_element_type=jnp.float32)
        + bn2_ref[...]
    )


def _node_mlp(x, parts, w1, bn1s, w2, bn2s):
    g = 5
    rb = N // g
    return pl.pallas_call(
        _node_body,
        grid=(g,),
        in_specs=[
            pl.BlockSpec((rb, H), lambda i: (i, 0)),
            pl.BlockSpec((NC, rb, H), lambda i: (0, i, 0)),
            pl.BlockSpec((2 * H, H), lambda i: (0, 0)),
            pl.BlockSpec((1, H), lambda i: (0, 0)),
            pl.BlockSpec((H, H), lambda i: (0, 0)),
            pl.BlockSpec((1, H), lambda i: (0, 0)),
        ],
        out_specs=pl.BlockSpec((rb, H), lambda i: (i, 0)),
        out_shape=jax.ShapeDtypeStruct((N, H), jnp.float32),
    )(x, parts, w1, bn1s.reshape(1, H), w2, bn2s.reshape(1, H))


def kernel(x, edge_index, edge_attr, We1, be1, We2, be2, Wn1, bn1, Wn2, bn2):
    src3 = edge_index[0].reshape(NW, NCH, CH)
    dst3 = edge_index[1].reshape(NW, NCH, CH)
    e = edge_attr
    n_steps = We1.shape[0]
    for s in range(n_steps):
        a = We1[s, :H]
        b = We1[s, H : 2 * H]
        c = We1[s, 2 * H :]
        p, q = _tc_xform(x, a, b, be1[s])
        t = _gather_t()(p, q, src3, dst3)
        e = _edge_mlp(t, e, c, We2[s], be2[s])
        parts = _scatter_agg()(e, dst3)
        x = _node_mlp(x, parts, Wn1[s], bn1[s], Wn2[s], bn2[s])
    return x, e
